# baseline (speedup 1.0000x reference)
import functools

import jax
import jax.numpy as jnp
import numpy as np
from jax import lax
from jax.experimental import pallas as pl
from jax.experimental.pallas import tpu as pltpu
from jax.experimental.pallas import tpu_sc as plsc

_STRIDES = (8, 16, 32)
_SIZES = ((64, 64), (32, 32), (16, 16))
_LROWS = (32, 8, 2)
_N = 5376
_ROWS = 42
_NCHUNK = _N // 16
_PRE_NMS_K = 1000
_MAX_DET = 100
_IOU_THR = 0.5
_SCORE_THR = 0.05
_B = 4
_CAP = 1024
_CLEN = _CAP + 80
_UNROLL = 2
_OLEN = _MAX_DET * 6
_OPAD = 640


def _location_consts():
    cxs, cys, svs = [], [], []
    for (h, w), s in zip(_SIZES, _STRIDES):
        ys = (np.arange(h, dtype=np.float32) + 0.5) * s
        xs = (np.arange(w, dtype=np.float32) + 0.5) * s
        cy, cx = np.meshgrid(ys, xs, indexing="ij")
        cxs.append(cx.reshape(-1))
        cys.append(cy.reshape(-1))
        svs.append(np.full(h * w, s, dtype=np.float32))
    cx = np.concatenate(cxs).reshape(_ROWS, 128)
    cy = np.concatenate(cys).reshape(_ROWS, 128)
    sv = np.concatenate(svs).reshape(_ROWS, 128)
    return cx, cy, sv


_CX, _CY, _SV = _location_consts()


def _decode_body(cls0_ref, cls1_ref, cls2_ref, reg0_ref, reg1_ref, reg2_ref,
                 cx_ref, cy_ref, sv_ref, out_ref):
    p = jnp.concatenate(
        [jax.nn.sigmoid(cls0_ref[...]),
         jax.nn.sigmoid(cls1_ref[...]),
         jax.nn.sigmoid(cls2_ref[...])], axis=2)

    score = p[:, 0]
    kind = jnp.zeros((_B, _ROWS, 128), jnp.float32)
    for c in range(1, 80):
        v = p[:, c]
        gt = v > score
        score = jnp.where(gt, v, score)
        kind = jnp.where(gt, jnp.float32(c), kind)

    cx = cx_ref[...][None]
    cy = cy_ref[...][None]
    sv = sv_ref[...][None]
    ltrb = jnp.concatenate(
        [jnp.exp(reg0_ref[...]), jnp.exp(reg1_ref[...]), jnp.exp(reg2_ref[...])],
        axis=2) * sv[:, None]
    x1 = cx - ltrb[:, 0]
    y1 = cy - ltrb[:, 1]
    x2 = cx + ltrb[:, 2]
    y2 = cy + ltrb[:, 3]
    areas = jnp.maximum(x2 - x1, 0.0) * jnp.maximum(y2 - y1, 0.0)

    s_pre = jnp.where(score > _SCORE_THR, score, -2.0)

    def bis(_, lohi):
        lo, hi = lohi
        mid = lo + (hi - lo + 1) // 2
        midf = lax.bitcast_convert_type(
            jnp.broadcast_to(mid, (_B, _ROWS, 128)), jnp.float32)
        c = jnp.sum((s_pre >= midf).astype(jnp.int32), axis=(1, 2), keepdims=True)
        ok = c >= _PRE_NMS_K
        return (jnp.where(ok, mid, lo), jnp.where(ok, hi, mid - 1))

    lo0 = jnp.zeros((_B, 1, 1), jnp.int32)
    hi0 = jnp.full((_B, 1, 1), 0x3F800000, jnp.int32)
    lo, _ = lax.fori_loop(0, 31, bis, (lo0, hi0))
    tf = lax.bitcast_convert_type(jnp.broadcast_to(lo, (_B, _ROWS, 128)), jnp.float32)
    s0 = jnp.where(s_pre >= tf, s_pre, -2.0)

    out_ref[:, 0] = s0
    out_ref[:, 1] = x1
    out_ref[:, 2] = y1
    out_ref[:, 3] = x2
    out_ref[:, 4] = y2
    out_ref[:, 5] = areas
    out_ref[:, 6] = kind
    out_ref[:, 7] = jnp.zeros((_B, _ROWS, 128), jnp.float32)


def _tc_decode(cls_l, reg_l):
    full = lambda shp: pl.BlockSpec(shp, lambda: (0,) * len(shp))
    return pl.pallas_call(
        _decode_body,
        in_specs=[full((_B, 80, r, 128)) for r in _LROWS]
        + [full((_B, 4, r, 128)) for r in _LROWS]
        + [full((_ROWS, 128))] * 3,
        out_specs=full((_B, 8, _ROWS, 128)),
        out_shape=jax.ShapeDtypeStruct((_B, 8, _ROWS, 128), jnp.float32),
    )(*cls_l, *reg_l, jnp.asarray(_CX), jnp.asarray(_CY), jnp.asarray(_SV))


def _sc_nms(data):
    mesh = plsc.VectorSubcoreMesh(core_axis_name="c", subcore_axis_name="s")

    @functools.partial(
        pl.kernel,
        mesh=mesh,
        compiler_params=pltpu.CompilerParams(needs_layout_passes=False),
        out_type=jax.ShapeDtypeStruct((_B, _OPAD), jnp.float32),
        scratch_types=[
            pltpu.VMEM((8, _N), jnp.float32),
            pltpu.VMEM((7, _CLEN), jnp.float32),
            pltpu.VMEM((_OPAD,), jnp.float32),
        ],
    )
    def nms_kernel(data_hbm, out_hbm, data_v, comp_v, outf_v):
        w = lax.axis_index("s") * 2 + lax.axis_index("c")

        @pl.when(w < _B)
        def _():
            pltpu.sync_copy(data_hbm.at[w], data_v)
            lane = lax.iota(jnp.int32, 16)
            zero16 = jnp.zeros((16,), jnp.float32)

            def zi(i, c):
                outf_v[pl.ds(i * 16, 16)] = zero16
                return c

            lax.fori_loop(0, _OPAD // 16, zi, 0)

            NEG = jnp.float32(-3.4e38)
            BIGI = jnp.int32(2 ** 30)

            def cmp_chunk(i, st):
                off_v, bv, bl = st
                base = i * 16
                s = data_v[0, pl.ds(base, 16)]
                msk = s > -1.0
                cum = plsc.cumsum(msk.astype(jnp.int32))
                pos = jnp.minimum(off_v + cum - 1, jnp.int32(_CAP + 15))
                plsc.store_scatter(comp_v, [jnp.zeros((16,), jnp.int32), pos],
                                   s, mask=msk)
                for r in range(1, 7):
                    v = data_v[r, pl.ds(base, 16)]
                    plsc.store_scatter(comp_v, [jnp.full((16,), r, jnp.int32), pos],
                                       v, mask=msk)
                gt = jnp.logical_and(msk, s > bv)
                bl = jnp.where(gt, pos, bl)
                bv = jnp.where(gt, s, bv)
                off_v = jnp.minimum(off_v + plsc.all_reduce_population_count(msk),
                                    jnp.int32(_CAP))
                return off_v, bv, bl

            off_v, bv, bl = lax.fori_loop(
                0, _NCHUNK, cmp_chunk,
                (jnp.zeros((16,), jnp.int32), jnp.full((16,), NEG),
                 jnp.zeros((16,), jnp.int32)))
            k = off_v[0]
            for u in range(_UNROLL):
                plsc.store_scatter(
                    comp_v, [jnp.zeros((16,), jnp.int32), k + u * 16 + lane],
                    jnp.full((16,), -2.0, jnp.float32))
            ng = (k + (_UNROLL * 16 - 1)) // (_UNROLL * 16)
            m0 = jnp.max(bv)
            idx0 = jnp.min(jnp.where(bv == m0, bl, BIGI))

            def step_body(st):
                cnt, m, idx = st
                rowv = jnp.minimum(lane, 6)
                idxv = jnp.full((16,), idx, jnp.int32)
                g = plsc.load_gather(comp_v, [rowv, idxv])
                bx1 = g[1]
                by1 = g[2]
                bx2 = g[3]
                by2 = g[4]
                ba = g[5]
                bk = g[6]
                row = jnp.where(lane == 0, bx1,
                      jnp.where(lane == 1, by1,
                      jnp.where(lane == 2, bx2,
                      jnp.where(lane == 3, by2,
                      jnp.where(lane == 4, bk,
                      jnp.where(lane == 5, m, 0.0))))))
                plsc.store_scatter(outf_v,
                                   [cnt * 6 + jnp.minimum(lane, jnp.int32(5))],
                                   row, mask=lane < 6)

                def sup_group(i, st2):
                    bv2, bl2 = st2
                    for u in range(_UNROLL):
                        base = (i * _UNROLL + u) * 16
                        s = comp_v[0, pl.ds(base, 16)]
                        x1 = comp_v[1, pl.ds(base, 16)]
                        y1 = comp_v[2, pl.ds(base, 16)]
                        x2 = comp_v[3, pl.ds(base, 16)]
                        y2 = comp_v[4, pl.ds(base, 16)]
                        ar = comp_v[5, pl.ds(base, 16)]
                        xx1 = jnp.maximum(bx1, x1)
                        yy1 = jnp.maximum(by1, y1)
                        xx2 = jnp.minimum(bx2, x2)
                        yy2 = jnp.minimum(by2, y2)
                        inter = (jnp.maximum(xx2 - xx1, 0.0)
                                 * jnp.maximum(yy2 - yy1, 0.0))
                        iou = inter / (ba + ar - inter + 1e-9)
                        s = jnp.where(iou > _IOU_THR, -2.0, s)
                        comp_v[0, pl.ds(base, 16)] = s
                        liv = base + lane
                        gt = s > bv2
                        bl2 = jnp.where(gt, liv, bl2)
                        bv2 = jnp.where(gt, s, bv2)
                    return bv2, bl2

                bv2, bl2 = lax.fori_loop(
                    0, ng, sup_group,
                    (jnp.full((16,), NEG), jnp.zeros((16,), jnp.int32)))
                m2 = jnp.max(bv2)
                idx2 = jnp.min(jnp.where(bv2 == m2, bl2, BIGI))
                return cnt + 1, m2, idx2

            def step(i, st):
                return lax.cond(st[1] > _SCORE_THR, step_body, lambda s: s, st)

            lax.fori_loop(0, _MAX_DET, step, (jnp.int32(0), m0, idx0))
            pltpu.sync_copy(outf_v, out_hbm.at[w])

    return nms_kernel(data)


def kernel(cls0, cls1, cls2, cnt0, cnt1, cnt2, reg0, reg1, reg2):
    del cnt0, cnt1, cnt2
    B = cls0.shape[0]
    cls_l = [c.reshape(B, 80, r, 128) for c, r in zip((cls0, cls1, cls2), _LROWS)]
    reg_l = [r_.reshape(B, 4, r, 128) for r_, r in zip((reg0, reg1, reg2), _LROWS)]

    data = _tc_decode(cls_l, reg_l).reshape(B, 8, _N)
    out = _sc_nms(data)
    return out[:, :_OLEN].reshape(B, _MAX_DET, 6)


# speedup vs baseline: 1.7243x; 1.7243x over previous
import functools

import jax
import jax.numpy as jnp
import numpy as np
from jax import lax
from jax.experimental import pallas as pl
from jax.experimental.pallas import tpu as pltpu
from jax.experimental.pallas import tpu_sc as plsc

_STRIDES = (8, 16, 32)
_SIZES = ((64, 64), (32, 32), (16, 16))
_LROWS = (32, 8, 2)
_N = 5376
_ROWS = 42
_NCHUNK = _N // 16
_PRE_NMS_K = 1000
_MAX_DET = 100
_IOU_THR = 0.5
_SCORE_THR = 0.05
_B = 4
_CAP = 1024
_CLEN = _CAP + 80
_UNROLL = 1
_OLEN = _MAX_DET * 6
_OPAD = 640


def _location_consts():
    cxs, cys, svs = [], [], []
    for (h, w), s in zip(_SIZES, _STRIDES):
        ys = (np.arange(h, dtype=np.float32) + 0.5) * s
        xs = (np.arange(w, dtype=np.float32) + 0.5) * s
        cy, cx = np.meshgrid(ys, xs, indexing="ij")
        cxs.append(cx.reshape(-1))
        cys.append(cy.reshape(-1))
        svs.append(np.full(h * w, s, dtype=np.float32))
    cx = np.concatenate(cxs).reshape(_ROWS, 128)
    cy = np.concatenate(cys).reshape(_ROWS, 128)
    sv = np.concatenate(svs).reshape(_ROWS, 128)
    return cx, cy, sv


_CX, _CY, _SV = _location_consts()


def _decode_body(cls0_ref, cls1_ref, cls2_ref, reg0_ref, reg1_ref, reg2_ref,
                 cx_ref, cy_ref, sv_ref, out_ref):
    p = jnp.concatenate(
        [jax.nn.sigmoid(cls0_ref[...]),
         jax.nn.sigmoid(cls1_ref[...]),
         jax.nn.sigmoid(cls2_ref[...])], axis=2)

    score = p[:, 0]
    kind = jnp.zeros((_B, _ROWS, 128), jnp.float32)
    for c in range(1, 80):
        v = p[:, c]
        gt = v > score
        score = jnp.where(gt, v, score)
        kind = jnp.where(gt, jnp.float32(c), kind)

    cx = cx_ref[...][None]
    cy = cy_ref[...][None]
    sv = sv_ref[...][None]
    ltrb = jnp.concatenate(
        [jnp.exp(reg0_ref[...]), jnp.exp(reg1_ref[...]), jnp.exp(reg2_ref[...])],
        axis=2) * sv[:, None]
    x1 = cx - ltrb[:, 0]
    y1 = cy - ltrb[:, 1]
    x2 = cx + ltrb[:, 2]
    y2 = cy + ltrb[:, 3]
    areas = jnp.maximum(x2 - x1, 0.0) * jnp.maximum(y2 - y1, 0.0)

    s_pre = jnp.where(score > _SCORE_THR, score, -2.0)

    def bis(_, lohi):
        lo, hi = lohi
        mid = lo + (hi - lo + 1) // 2
        midf = lax.bitcast_convert_type(
            jnp.broadcast_to(mid, (_B, _ROWS, 128)), jnp.float32)
        c = jnp.sum((s_pre >= midf).astype(jnp.int32), axis=(1, 2), keepdims=True)
        ok = c >= _PRE_NMS_K
        return (jnp.where(ok, mid, lo), jnp.where(ok, hi, mid - 1))

    lo0 = jnp.zeros((_B, 1, 1), jnp.int32)
    hi0 = jnp.full((_B, 1, 1), 0x3F800000, jnp.int32)
    lo, _ = lax.fori_loop(0, 31, bis, (lo0, hi0))
    tf = lax.bitcast_convert_type(jnp.broadcast_to(lo, (_B, _ROWS, 128)), jnp.float32)
    s0 = jnp.where(s_pre >= tf, s_pre, -2.0)

    out_ref[:, 0] = s0
    out_ref[:, 1] = x1
    out_ref[:, 2] = y1
    out_ref[:, 3] = x2
    out_ref[:, 4] = y2
    out_ref[:, 5] = areas
    out_ref[:, 6] = kind
    out_ref[:, 7] = jnp.zeros((_B, _ROWS, 128), jnp.float32)


def _tc_decode(cls_l, reg_l):
    full = lambda shp: pl.BlockSpec(shp, lambda: (0,) * len(shp))
    return pl.pallas_call(
        _decode_body,
        in_specs=[full((_B, 80, r, 128)) for r in _LROWS]
        + [full((_B, 4, r, 128)) for r in _LROWS]
        + [full((_ROWS, 128))] * 3,
        out_specs=full((_B, 8, _ROWS, 128)),
        out_shape=jax.ShapeDtypeStruct((_B, 8, _ROWS, 128), jnp.float32),
    )(*cls_l, *reg_l, jnp.asarray(_CX), jnp.asarray(_CY), jnp.asarray(_SV))


def _sc_nms(data):
    mesh = plsc.VectorSubcoreMesh(core_axis_name="c", subcore_axis_name="s")

    @functools.partial(
        pl.kernel,
        mesh=mesh,
        compiler_params=pltpu.CompilerParams(needs_layout_passes=False),
        out_type=jax.ShapeDtypeStruct((_B, _OPAD), jnp.float32),
        scratch_types=[
            pltpu.VMEM((8, _N), jnp.float32),
            pltpu.VMEM((7, _CLEN), jnp.float32),
            pltpu.VMEM((_OPAD,), jnp.float32),
        ],
    )
    def nms_kernel(data_hbm, out_hbm, data_v, comp_v, outf_v):
        w = lax.axis_index("s") * 2 + lax.axis_index("c")

        @pl.when(w < _B)
        def _():
            pltpu.sync_copy(data_hbm.at[w], data_v)
            lane = lax.iota(jnp.int32, 16)
            zero16 = jnp.zeros((16,), jnp.float32)

            def zi(i, c):
                outf_v[pl.ds(i * 16, 16)] = zero16
                return c

            lax.fori_loop(0, _OPAD // 16, zi, 0)

            NEG = jnp.float32(-3.4e38)
            BIGI = jnp.int32(2 ** 30)

            def cmp_chunk(i, st):
                off_v, bv, bl = st
                base = i * 16
                s = data_v[0, pl.ds(base, 16)]
                msk = s > -1.0
                cum = plsc.cumsum(msk.astype(jnp.int32))
                pos = jnp.minimum(off_v + cum - 1, jnp.int32(_CAP + 15))
                plsc.store_scatter(comp_v, [jnp.zeros((16,), jnp.int32), pos],
                                   s, mask=msk)
                for r in range(1, 7):
                    v = data_v[r, pl.ds(base, 16)]
                    plsc.store_scatter(comp_v, [jnp.full((16,), r, jnp.int32), pos],
                                       v, mask=msk)
                gt = jnp.logical_and(msk, s > bv)
                bl = jnp.where(gt, pos, bl)
                bv = jnp.where(gt, s, bv)
                off_v = jnp.minimum(off_v + plsc.all_reduce_population_count(msk),
                                    jnp.int32(_CAP))
                return off_v, bv, bl

            off_v, bv, bl = lax.fori_loop(
                0, _NCHUNK, cmp_chunk,
                (jnp.zeros((16,), jnp.int32), jnp.full((16,), NEG),
                 jnp.zeros((16,), jnp.int32)))
            k = off_v[0]
            for u in range(_UNROLL):
                plsc.store_scatter(
                    comp_v, [jnp.zeros((16,), jnp.int32), k + u * 16 + lane],
                    jnp.full((16,), -2.0, jnp.float32))
            ng = (k + (_UNROLL * 16 - 1)) // (_UNROLL * 16)
            m0 = jnp.max(bv)
            idx0 = jnp.min(jnp.where(bv == m0, bl, BIGI))

            def step_body(st):
                cnt, m, idx = st
                rowv = jnp.minimum(lane, 6)
                idxv = jnp.full((16,), idx, jnp.int32)
                g = plsc.load_gather(comp_v, [rowv, idxv])
                bx1 = g[1]
                by1 = g[2]
                bx2 = g[3]
                by2 = g[4]
                ba = g[5]
                bk = g[6]
                row = jnp.where(lane == 0, bx1,
                      jnp.where(lane == 1, by1,
                      jnp.where(lane == 2, bx2,
                      jnp.where(lane == 3, by2,
                      jnp.where(lane == 4, bk,
                      jnp.where(lane == 5, m, 0.0))))))
                plsc.store_scatter(outf_v,
                                   [cnt * 6 + jnp.minimum(lane, jnp.int32(5))],
                                   row, mask=lane < 6)

                def sup_group(i, st2):
                    bv2, bl2 = st2
                    for u in range(_UNROLL):
                        base = (i * _UNROLL + u) * 16
                        s = comp_v[0, pl.ds(base, 16)]
                        x1 = comp_v[1, pl.ds(base, 16)]
                        y1 = comp_v[2, pl.ds(base, 16)]
                        x2 = comp_v[3, pl.ds(base, 16)]
                        y2 = comp_v[4, pl.ds(base, 16)]
                        ar = comp_v[5, pl.ds(base, 16)]
                        xx1 = jnp.maximum(bx1, x1)
                        yy1 = jnp.maximum(by1, y1)
                        xx2 = jnp.minimum(bx2, x2)
                        yy2 = jnp.minimum(by2, y2)
                        inter = (jnp.maximum(xx2 - xx1, 0.0)
                                 * jnp.maximum(yy2 - yy1, 0.0))
                        iou = inter / (ba + ar - inter + 1e-9)
                        s = jnp.where(iou > _IOU_THR, -2.0, s)
                        comp_v[0, pl.ds(base, 16)] = s
                        liv = base + lane
                        gt = s > bv2
                        bl2 = jnp.where(gt, liv, bl2)
                        bv2 = jnp.where(gt, s, bv2)
                    return bv2, bl2

                bv2, bl2 = lax.fori_loop(
                    0, ng, sup_group,
                    (jnp.full((16,), NEG), jnp.zeros((16,), jnp.int32)))
                m2 = jnp.max(bv2)
                idx2 = jnp.min(jnp.where(bv2 == m2, bl2, BIGI))
                return cnt + 1, m2, idx2

            def step(i, st):
                return lax.cond(st[1] > _SCORE_THR, step_body, lambda s: s, st)

            lax.fori_loop(0, _MAX_DET, step, (jnp.int32(0), m0, idx0))
            pltpu.sync_copy(outf_v, out_hbm.at[w])

    return nms_kernel(data)


def kernel(cls0, cls1, cls2, cnt0, cnt1, cnt2, reg0, reg1, reg2):
    del cnt0, cnt1, cnt2
    B = cls0.shape[0]
    cls_l = [c.reshape(B, 80, r, 128) for c, r in zip((cls0, cls1, cls2), _LROWS)]
    reg_l = [r_.reshape(B, 4, r, 128) for r_, r in zip((reg0, reg1, reg2), _LROWS)]

    data = _tc_decode(cls_l, reg_l).reshape(B, 8, _N)
    out = _sc_nms(data)
    return out[:, :_OLEN].reshape(B, _MAX_DET, 6)
